# Initial kernel scaffold; baseline (speedup 1.0000x reference)
import jax
import jax.numpy as jnp
from jax.experimental import pallas as pl


def kernel(inputs):
    raise NotImplementedError("write your pallas kernel here")


# speedup vs baseline: 40.6616x; 40.6616x over previous
import functools

import jax
import jax.numpy as jnp
from jax.experimental import pallas as pl
from jax.experimental.pallas import tpu as pltpu

_K = 8
_NEG = float("-inf")


def _body(x_ref, o_ref, acc_ref, *, nvec, nchunks):
    j = pl.program_id(1)

    @pl.when(j == 0)
    def _init():
        acc_ref[...] = jnp.full((_K, 8, 128), _NEG, jnp.float32)

    carry = tuple(acc_ref[i] for i in range(_K))

    def step(i, ms):
        ms = list(ms)
        for u in range(4):
            v = x_ref[0, pl.ds((i * 4 + u) * 8, 8), :]
            for lvl in range(_K):
                hi = jnp.maximum(ms[lvl], v)
                v = jnp.minimum(ms[lvl], v)
                ms[lvl] = hi
        return tuple(ms)

    carry = jax.lax.fori_loop(0, nvec // 4, step, carry)
    for i in range(_K):
        acc_ref[i] = carry[i]

    @pl.when(j == nchunks - 1)
    def _finalize():
        ms = list(carry)
        lane = jax.lax.broadcasted_iota(jnp.int32, (8, 128), 1)
        sub = jax.lax.broadcasted_iota(jnp.int32, (8, 128), 0)
        half = lane // 64
        big = jnp.int32(1 << 20)
        rows = []
        for _ in range(_K):
            cur = functools.reduce(jnp.maximum, ms)
            colmax = jnp.max(cur, axis=0, keepdims=True)
            fold = jnp.maximum(colmax, pltpu.roll(colmax, 64, 1))
            rows.append(fold)
            bc = jnp.broadcast_to(fold, (8, 128))
            idxs = [
                jnp.where(ms[i] == bc, i * 16 + sub * 2 + half, big)
                for i in range(_K)
            ]
            mini = functools.reduce(jnp.minimum, idxs)
            mcol = jnp.min(mini, axis=0, keepdims=True)
            mfold = jnp.minimum(mcol, pltpu.roll(mcol, 64, 1))
            mbc = jnp.broadcast_to(mfold, (8, 128))
            ms = [
                jnp.where(
                    (ms[i] == bc) & ((i * 16 + sub * 2 + half) == mbc),
                    _NEG,
                    ms[i],
                )
                for i in range(_K)
            ]
        o_ref[0] = jnp.concatenate(rows, axis=0)


def kernel(inputs):
    b, s, c = inputs.shape
    s2 = s * c // 128
    x2 = inputs.reshape(b, s2, 128)
    rows = 2048 if s2 % 2048 == 0 else s2
    nchunks = s2 // rows

    out = pl.pallas_call(
        functools.partial(_body, nvec=rows // 8, nchunks=nchunks),
        grid=(b, nchunks),
        in_specs=[pl.BlockSpec((1, rows, 128), lambda bi, ji: (bi, ji, 0))],
        out_specs=pl.BlockSpec((1, _K, 128), lambda bi, ji: (bi, 0, 0)),
        out_shape=jax.ShapeDtypeStruct((b, _K, 128), jnp.float32),
        scratch_shapes=[pltpu.VMEM((_K, 8, 128), jnp.float32)],
        compiler_params=pltpu.CompilerParams(
            dimension_semantics=("arbitrary", "arbitrary")
        ),
    )(x2)
    return out[:, :, : c].transpose(0, 2, 1).reshape(b, c * _K)


# speedup vs baseline: 41.8372x; 1.0289x over previous
import functools

import jax
import jax.numpy as jnp
from jax.experimental import pallas as pl
from jax.experimental.pallas import tpu as pltpu

_K = 8
_NEG = float("-inf")
_D8 = 8
_D4 = 4


def _insert(levels, v):
    out = []
    for m in levels:
        out.append(jnp.maximum(m, v))
        v = jnp.minimum(m, v)
    return out


def _body(x_ref, o_ref, acc_ref, *, nvec, nchunks):
    j = pl.program_id(1)
    nlev = 2 * (_D8 + _D4)

    @pl.when(j == 0)
    def _init():
        acc_ref[...] = jnp.full((nlev, 8, 128), _NEG, jnp.float32)

    groups = []
    flat = [acc_ref[i] for i in range(nlev)]

    def unflatten(f):
        return [
            (f[0:_D8], f[_D8 : _D8 + _D4]),
            (f[_D8 + _D4 : 2 * _D8 + _D4], f[2 * _D8 + _D4 :]),
        ]

    def flatten(gs):
        out = []
        for ms, mn in gs:
            out.extend(ms)
            out.extend(mn)
        return tuple(out)

    def step(i, f):
        gs = unflatten(list(f))
        base = i * 8
        for p in range(4):
            va = x_ref[0, pl.ds((base + 2 * p) * 8, 8), :]
            vb = x_ref[0, pl.ds((base + 2 * p + 1) * 8, 8), :]
            hi = jnp.maximum(va, vb)
            lo = jnp.minimum(va, vb)
            ms, mn = gs[p % 2]
            gs[p % 2] = (_insert(ms, hi), _insert(mn, lo))
        return flatten(gs)

    flat = jax.lax.fori_loop(0, nvec // 8, step, tuple(flat))
    for i in range(nlev):
        acc_ref[i] = flat[i]

    @pl.when(j == nchunks - 1)
    def _finalize():
        ms = list(flat)
        lane = jax.lax.broadcasted_iota(jnp.int32, (8, 128), 1)
        sub = jax.lax.broadcasted_iota(jnp.int32, (8, 128), 0)
        half = lane // 64
        big = jnp.int32(1 << 20)
        rows = []
        for _ in range(_K):
            cur = functools.reduce(jnp.maximum, ms)
            colmax = jnp.max(cur, axis=0, keepdims=True)
            fold = jnp.maximum(colmax, pltpu.roll(colmax, 64, 1))
            rows.append(fold)
            bc = jnp.broadcast_to(fold, (8, 128))
            idxs = [
                jnp.where(ms[i] == bc, i * 16 + sub * 2 + half, big)
                for i in range(nlev)
            ]
            mini = functools.reduce(jnp.minimum, idxs)
            mcol = jnp.min(mini, axis=0, keepdims=True)
            mfold = jnp.minimum(mcol, pltpu.roll(mcol, 64, 1))
            mbc = jnp.broadcast_to(mfold, (8, 128))
            ms = [
                jnp.where(
                    (ms[i] == bc) & ((i * 16 + sub * 2 + half) == mbc),
                    _NEG,
                    ms[i],
                )
                for i in range(nlev)
            ]
        o_ref[0] = jnp.concatenate(rows, axis=0)


def kernel(inputs):
    b, s, c = inputs.shape
    s2 = s * c // 128
    x2 = inputs.reshape(b, s2, 128)
    rows = 2048 if s2 % 2048 == 0 else s2
    nchunks = s2 // rows

    out = pl.pallas_call(
        functools.partial(_body, nvec=rows // 8, nchunks=nchunks),
        grid=(b, nchunks),
        in_specs=[pl.BlockSpec((1, rows, 128), lambda bi, ji: (bi, ji, 0))],
        out_specs=pl.BlockSpec((1, _K, 128), lambda bi, ji: (bi, 0, 0)),
        out_shape=jax.ShapeDtypeStruct((b, _K, 128), jnp.float32),
        scratch_shapes=[pltpu.VMEM((2 * (_D8 + _D4), 8, 128), jnp.float32)],
        compiler_params=pltpu.CompilerParams(
            dimension_semantics=("arbitrary", "arbitrary")
        ),
    )(x2)
    return out[:, :, : c].transpose(0, 2, 1).reshape(b, c * _K)
